# baseline (speedup 1.0000x reference)
import functools

import jax
import jax.numpy as jnp
from jax.experimental import pallas as pl
from jax.experimental.pallas import tpu as pltpu

_THRESHOLD = 64.0
_NEG = -1e30
_LANES = 128


def _block_top2(read_chunk, nchunks):
    bm1 = read_chunk(0)
    bm2 = jnp.full_like(bm1, _NEG)
    for k in range(1, nchunks):
        xk = read_chunk(k)
        bm2 = jnp.maximum(bm2, jnp.minimum(bm1, xk))
        bm1 = jnp.maximum(bm1, xk)
    return bm1, bm2


def _merge_top2(a1, a2, b1, b2):
    m1 = jnp.maximum(a1, b1)
    m2 = jnp.maximum(jnp.minimum(a1, b1), jnp.where(a1 >= b1, a2, b2))
    return m1, m2


def _tight_closs_kernel(out_mat, true_ref, res_ref, m1_ref, m2_ref, s_ref,
                        *, blk, ncols, nblocks):
    j = pl.program_id(0)
    nchunks = blk // _LANES

    @pl.when(j == 0)
    def _init():
        m1_ref[...] = jnp.full_like(m1_ref, _NEG)
        m2_ref[...] = jnp.full_like(m2_ref, _NEG)
        s_ref[...] = jnp.zeros_like(s_ref)

    def _process(read_chunk):
        bm1, bm2 = _block_top2(read_chunk, nchunks)
        a1, a2 = m1_ref[...], m2_ref[...]
        m1n, m2n = _merge_top2(a1, a2, bm1, bm2)
        m1_ref[...] = m1n
        m2_ref[...] = m2n
        es = s_ref[...] * jnp.exp(a1 - m1n)
        for k in range(nchunks):
            es = es + jnp.exp(read_chunk(k) - m1n)
        s_ref[...] = es

    @pl.when(j < nblocks - 1)
    def _steady():
        _process(lambda k: out_mat[:, k * _LANES:(k + 1) * _LANES])

    @pl.when(j == nblocks - 1)
    def _last():
        base = j * blk
        civ = jax.lax.broadcasted_iota(jnp.int32, (128, _LANES), 1)

        def _read_masked(k):
            xk = out_mat[:, k * _LANES:(k + 1) * _LANES]
            return jnp.where(base + k * _LANES + civ < ncols, xk, _NEG)

        _process(_read_masked)

        m1, m2 = m1_ref[...], m2_ref[...]
        sh = _LANES
        while sh > 1:
            sh //= 2
            b1 = pltpu.roll(m1, sh, 1)
            b2 = pltpu.roll(m2, sh, 1)
            m1, m2 = _merge_top2(m1, m2, b1, b2)
        row_m1 = jnp.max(m1_ref[...], axis=1, keepdims=True)
        row_m2 = m2[:, 0:1]
        s = s_ref[...]
        row_s = jnp.sum(s * jnp.exp(m1_ref[...] - row_m1), axis=1,
                        keepdims=True)

        true = true_ref[...]
        masked_max = jnp.where(true == row_m1, row_m2, row_m1)
        margin = true - masked_max
        lse = row_m1 + jnp.log(row_s)
        l = jnp.where(margin >= 0.0, 1.0 - margin, 1.0 - true + lse)
        l = jnp.maximum(l, 0.0)

        ones_row = jnp.ones((1, _LANES), jnp.float32)
        bc = jax.lax.dot_general(l, ones_row, (((1,), (0,)), ((), ())),
                                 precision=jax.lax.Precision.HIGHEST)
        br = bc.T
        ii = jax.lax.broadcasted_iota(jnp.int32, (_LANES, _LANES), 0)
        jj = jax.lax.broadcasted_iota(jnp.int32, (_LANES, _LANES), 1)
        prec = ((br < bc) | ((br == bc) & (jj < ii))).astype(jnp.float32)
        incl = jnp.where((br == bc) & (jj == ii), 1.0, prec)
        ones_col = jnp.ones((_LANES, 1), jnp.float32)
        rank = jax.lax.dot_general(prec, ones_col, (((1,), (0,)), ((), ())),
                                   precision=jax.lax.Precision.HIGHEST)
        csum = jax.lax.dot_general(incl, l, (((1,), (0,)), ((), ())),
                                   precision=jax.lax.Precision.HIGHEST)
        keep = (csum <= _THRESHOLD + 1.0 - rank).astype(jnp.float32)
        c1 = jnp.sum(keep * l)
        c2 = jnp.float32(_LANES) - jnp.sum(keep)
        res_ref[0, 0] = jnp.where(c1 < c2, c2, c1)


@jax.jit
def kernel(output, target):
    B, V = output.shape
    blk = 2048
    nblocks = pl.cdiv(V, blk)
    rows = jnp.arange(B, dtype=jnp.int32)
    true = output[rows, target.astype(jnp.int32)].reshape(B, 1)

    res = pl.pallas_call(
        functools.partial(_tight_closs_kernel, blk=blk, ncols=V,
                          nblocks=nblocks),
        grid=(nblocks,),
        in_specs=[
            pl.BlockSpec((B, blk), lambda j: (0, j)),
            pl.BlockSpec((B, 1), lambda j: (0, 0)),
        ],
        out_specs=pl.BlockSpec((1, 1), lambda j: (0, 0),
                               memory_space=pltpu.SMEM),
        out_shape=jax.ShapeDtypeStruct((1, 1), jnp.float32),
        scratch_shapes=[
            pltpu.VMEM((B, _LANES), jnp.float32),
            pltpu.VMEM((B, _LANES), jnp.float32),
            pltpu.VMEM((B, _LANES), jnp.float32),
        ],
    )(output, true)
    return res[0, 0]


# speedup vs baseline: 1.1535x; 1.1535x over previous
import functools

import jax
import jax.numpy as jnp
from jax.experimental import pallas as pl
from jax.experimental.pallas import tpu as pltpu

_THRESHOLD = 64.0
_NEG = -1e30
_LANES = 128


def _block_top2(read_chunk, nchunks):
    bm1 = read_chunk(0)
    bm2 = jnp.full_like(bm1, _NEG)
    for k in range(1, nchunks):
        xk = read_chunk(k)
        bm2 = jnp.maximum(bm2, jnp.minimum(bm1, xk))
        bm1 = jnp.maximum(bm1, xk)
    return bm1, bm2


def _merge_top2(a1, a2, b1, b2):
    m1 = jnp.maximum(a1, b1)
    m2 = jnp.maximum(jnp.minimum(a1, b1), jnp.where(a1 >= b1, a2, b2))
    return m1, m2


def _tight_closs_kernel(out_mat, true_ref, res_ref, m1_ref, m2_ref, s_ref,
                        *, blk, ncols, nblocks):
    j = pl.program_id(0)
    nchunks = blk // _LANES

    @pl.when(j == 0)
    def _init():
        m1_ref[...] = jnp.full_like(m1_ref, _NEG)
        m2_ref[...] = jnp.full_like(m2_ref, _NEG)
        s_ref[...] = jnp.zeros_like(s_ref)

    def _process(read_chunk):
        bm1, bm2 = _block_top2(read_chunk, nchunks)
        a1, a2 = m1_ref[...], m2_ref[...]
        m1n, m2n = _merge_top2(a1, a2, bm1, bm2)
        m1_ref[...] = m1n
        m2_ref[...] = m2n
        es = s_ref[...] * jnp.exp(a1 - m1n)
        for k in range(nchunks):
            es = es + jnp.exp(read_chunk(k) - m1n)
        s_ref[...] = es

    @pl.when(j < nblocks - 1)
    def _steady():
        _process(lambda k: out_mat[:, k * _LANES:(k + 1) * _LANES])

    @pl.when(j == nblocks - 1)
    def _last():
        base = j * blk
        civ = jax.lax.broadcasted_iota(jnp.int32, (128, _LANES), 1)

        def _read_masked(k):
            xk = out_mat[:, k * _LANES:(k + 1) * _LANES]
            return jnp.where(base + k * _LANES + civ < ncols, xk, _NEG)

        _process(_read_masked)

        m1, m2 = m1_ref[...], m2_ref[...]
        sh = _LANES
        while sh > 1:
            sh //= 2
            b1 = pltpu.roll(m1, sh, 1)
            b2 = pltpu.roll(m2, sh, 1)
            m1, m2 = _merge_top2(m1, m2, b1, b2)
        row_m1 = jnp.max(m1_ref[...], axis=1, keepdims=True)
        row_m2 = m2[:, 0:1]
        s = s_ref[...]
        row_s = jnp.sum(s * jnp.exp(m1_ref[...] - row_m1), axis=1,
                        keepdims=True)

        true = true_ref[...]
        masked_max = jnp.where(true == row_m1, row_m2, row_m1)
        margin = true - masked_max
        lse = row_m1 + jnp.log(row_s)
        l = jnp.where(margin >= 0.0, 1.0 - margin, 1.0 - true + lse)
        l = jnp.maximum(l, 0.0)

        ones_row = jnp.ones((1, _LANES), jnp.float32)
        bc = jax.lax.dot_general(l, ones_row, (((1,), (0,)), ((), ())),
                                 precision=jax.lax.Precision.HIGHEST)
        br = bc.T
        ii = jax.lax.broadcasted_iota(jnp.int32, (_LANES, _LANES), 0)
        jj = jax.lax.broadcasted_iota(jnp.int32, (_LANES, _LANES), 1)
        prec = ((br < bc) | ((br == bc) & (jj < ii))).astype(jnp.float32)
        incl = jnp.where((br == bc) & (jj == ii), 1.0, prec)
        ones_col = jnp.ones((_LANES, 1), jnp.float32)
        rank = jax.lax.dot_general(prec, ones_col, (((1,), (0,)), ((), ())),
                                   precision=jax.lax.Precision.HIGHEST)
        csum = jax.lax.dot_general(incl, l, (((1,), (0,)), ((), ())),
                                   precision=jax.lax.Precision.HIGHEST)
        keep = (csum <= _THRESHOLD + 1.0 - rank).astype(jnp.float32)
        c1 = jnp.sum(keep * l)
        c2 = jnp.float32(_LANES) - jnp.sum(keep)
        res_ref[0, 0] = jnp.where(c1 < c2, c2, c1)


@jax.jit
def kernel(output, target):
    B, V = output.shape
    blk = 4096
    nblocks = pl.cdiv(V, blk)
    rows = jnp.arange(B, dtype=jnp.int32)
    true = output[rows, target.astype(jnp.int32)].reshape(B, 1)

    res = pl.pallas_call(
        functools.partial(_tight_closs_kernel, blk=blk, ncols=V,
                          nblocks=nblocks),
        grid=(nblocks,),
        in_specs=[
            pl.BlockSpec((B, blk), lambda j: (0, j)),
            pl.BlockSpec((B, 1), lambda j: (0, 0)),
        ],
        out_specs=pl.BlockSpec((1, 1), lambda j: (0, 0),
                               memory_space=pltpu.SMEM),
        out_shape=jax.ShapeDtypeStruct((1, 1), jnp.float32),
        scratch_shapes=[
            pltpu.VMEM((B, _LANES), jnp.float32),
            pltpu.VMEM((B, _LANES), jnp.float32),
            pltpu.VMEM((B, _LANES), jnp.float32),
        ],
    )(output, true)
    return res[0, 0]


# speedup vs baseline: 1.2422x; 1.0769x over previous
import functools

import jax
import jax.numpy as jnp
from jax.experimental import pallas as pl
from jax.experimental.pallas import tpu as pltpu

_THRESHOLD = 64.0
_NEG = -1e30
_LANES = 128


def _block_top2(read_chunk, nchunks):
    bm1 = read_chunk(0)
    bm2 = jnp.full_like(bm1, _NEG)
    for k in range(1, nchunks):
        xk = read_chunk(k)
        bm2 = jnp.maximum(bm2, jnp.minimum(bm1, xk))
        bm1 = jnp.maximum(bm1, xk)
    return bm1, bm2


def _merge_top2(a1, a2, b1, b2):
    m1 = jnp.maximum(a1, b1)
    m2 = jnp.maximum(jnp.minimum(a1, b1), jnp.where(a1 >= b1, a2, b2))
    return m1, m2


def _tight_closs_kernel(out_mat, true_ref, res_ref, m1_ref, m2_ref, s_ref,
                        *, blk, ncols, nblocks):
    j = pl.program_id(0)
    nchunks = blk // _LANES

    @pl.when(j == 0)
    def _init():
        m1_ref[...] = jnp.full_like(m1_ref, _NEG)
        m2_ref[...] = jnp.full_like(m2_ref, _NEG)
        s_ref[...] = jnp.zeros_like(s_ref)

    def _process(read_chunk):
        bm1, bm2 = _block_top2(read_chunk, nchunks)
        a1, a2 = m1_ref[...], m2_ref[...]
        m1n, m2n = _merge_top2(a1, a2, bm1, bm2)
        m1_ref[...] = m1n
        m2_ref[...] = m2n
        es = s_ref[...] * jnp.exp(a1 - m1n)
        for k in range(nchunks):
            es = es + jnp.exp(read_chunk(k) - m1n)
        s_ref[...] = es

    @pl.when(j < nblocks - 1)
    def _steady():
        _process(lambda k: out_mat[:, k * _LANES:(k + 1) * _LANES])

    @pl.when(j == nblocks - 1)
    def _last():
        base = j * blk
        civ = jax.lax.broadcasted_iota(jnp.int32, (128, _LANES), 1)

        def _read_masked(k):
            xk = out_mat[:, k * _LANES:(k + 1) * _LANES]
            return jnp.where(base + k * _LANES + civ < ncols, xk, _NEG)

        _process(_read_masked)

        m1, m2 = m1_ref[...], m2_ref[...]
        sh = _LANES
        while sh > 1:
            sh //= 2
            b1 = pltpu.roll(m1, sh, 1)
            b2 = pltpu.roll(m2, sh, 1)
            m1, m2 = _merge_top2(m1, m2, b1, b2)
        row_m1 = jnp.max(m1_ref[...], axis=1, keepdims=True)
        row_m2 = m2[:, 0:1]
        s = s_ref[...]
        row_s = jnp.sum(s * jnp.exp(m1_ref[...] - row_m1), axis=1,
                        keepdims=True)

        true = true_ref[...]
        masked_max = jnp.where(true == row_m1, row_m2, row_m1)
        margin = true - masked_max
        lse = row_m1 + jnp.log(row_s)
        l = jnp.where(margin >= 0.0, 1.0 - margin, 1.0 - true + lse)
        l = jnp.maximum(l, 0.0)

        ones_row = jnp.ones((1, _LANES), jnp.float32)
        bc = jax.lax.dot_general(l, ones_row, (((1,), (0,)), ((), ())),
                                 precision=jax.lax.Precision.HIGHEST)
        br = bc.T
        ii = jax.lax.broadcasted_iota(jnp.int32, (_LANES, _LANES), 0)
        jj = jax.lax.broadcasted_iota(jnp.int32, (_LANES, _LANES), 1)
        prec = ((br < bc) | ((br == bc) & (jj < ii))).astype(jnp.float32)
        incl = jnp.where((br == bc) & (jj == ii), 1.0, prec)
        ones_col = jnp.ones((_LANES, 1), jnp.float32)
        rank = jax.lax.dot_general(prec, ones_col, (((1,), (0,)), ((), ())),
                                   precision=jax.lax.Precision.HIGHEST)
        csum = jax.lax.dot_general(incl, l, (((1,), (0,)), ((), ())),
                                   precision=jax.lax.Precision.HIGHEST)
        keep = (csum <= _THRESHOLD + 1.0 - rank).astype(jnp.float32)
        c1 = jnp.sum(keep * l)
        c2 = jnp.float32(_LANES) - jnp.sum(keep)
        res_ref[0, 0] = jnp.where(c1 < c2, c2, c1)


@jax.jit
def kernel(output, target):
    B, V = output.shape
    blk = 8192
    nblocks = pl.cdiv(V, blk)
    rows = jnp.arange(B, dtype=jnp.int32)
    true = output[rows, target.astype(jnp.int32)].reshape(B, 1)

    res = pl.pallas_call(
        functools.partial(_tight_closs_kernel, blk=blk, ncols=V,
                          nblocks=nblocks),
        grid=(nblocks,),
        in_specs=[
            pl.BlockSpec((B, blk), lambda j: (0, j)),
            pl.BlockSpec((B, 1), lambda j: (0, 0)),
        ],
        out_specs=pl.BlockSpec((1, 1), lambda j: (0, 0),
                               memory_space=pltpu.SMEM),
        out_shape=jax.ShapeDtypeStruct((1, 1), jnp.float32),
        scratch_shapes=[
            pltpu.VMEM((B, _LANES), jnp.float32),
            pltpu.VMEM((B, _LANES), jnp.float32),
            pltpu.VMEM((B, _LANES), jnp.float32),
        ],
    )(output, true)
    return res[0, 0]
